# Initial kernel scaffold; baseline (speedup 1.0000x reference)
import jax
import jax.numpy as jnp
from jax.experimental import pallas as pl


def kernel(index, node_keys, edge_index, W, M, Internal_Node_Impact):
    raise NotImplementedError("write your pallas kernel here")


# speedup vs baseline: 5.2884x; 5.2884x over previous
import functools

import jax
import jax.numpy as jnp
from jax import lax
from jax.experimental import pallas as pl
from jax.experimental.pallas import tpu as pltpu
from jax.experimental.pallas import tpu_sc as plsc

N = 10000
E = 320000
D = 128
K = 10000

NC = 2
NS = 16
NP = 10240
CH = 80
EPT = E // (NC * NS)
NCH = EPT // CH
KPT = NP // (NC * NS)
RPT = NP // NS
ZR = 128


def _sc_stage(table, keys_pad, src, dst, zeros_blk):
    mesh = plsc.VectorSubcoreMesh(
        core_axis_name="c", subcore_axis_name="s",
        num_cores=NC, num_subcores=NS)

    @functools.partial(
        pl.kernel,
        out_type=[
            jax.ShapeDtypeStruct((NP, D), jnp.float32),
            jax.ShapeDtypeStruct((NC, NP, D), jnp.float32),
        ],
        mesh=mesh,
        scratch_types=[
            pltpu.VMEM((CH,), jnp.int32),
            pltpu.VMEM((CH,), jnp.int32),
            pltpu.VMEM((CH, D), jnp.float32),
            pltpu.VMEM((ZR, D), jnp.float32),
            pltpu.VMEM_SHARED((NP, D), jnp.float32),
            pltpu.SemaphoreType.DMA,
        ],
    )
    def sc_kernel(table_hbm, keys_hbm, src_hbm, dst_hbm, z_hbm,
                  gs_hbm, parts_hbm,
                  idx_v, dst_v, rows_v, zb_v, acc_sh, sem):
        cid = lax.axis_index("c")
        sid = lax.axis_index("s")
        wid = cid * NS + sid

        kbase = wid * KPT
        for b in range(KPT // CH):
            pltpu.sync_copy(keys_hbm.at[pl.ds(kbase + b * CH, CH)], idx_v)
            pltpu.async_copy(table_hbm.at[idx_v], rows_v, sem).wait()
            pltpu.sync_copy(rows_v, gs_hbm.at[pl.ds(kbase + b * CH, CH)])

        pltpu.sync_copy(z_hbm, zb_v)
        zbase = sid * RPT
        for b in range(RPT // ZR):
            pltpu.sync_copy(zb_v, acc_sh.at[pl.ds(zbase + b * ZR, ZR)])
        plsc.subcore_barrier()

        ebase = cid * (E // NC) + sid * EPT

        def body(i, carry):
            base = ebase + i * CH
            pltpu.sync_copy(src_hbm.at[pl.ds(base, CH)], idx_v)
            pltpu.async_copy(table_hbm.at[idx_v], rows_v, sem).wait()
            pltpu.sync_copy(dst_hbm.at[pl.ds(base, CH)], dst_v)
            pltpu.sync_copy(rows_v, acc_sh.at[dst_v], add=True)
            return carry

        lax.fori_loop(0, NCH, body, 0)
        plsc.subcore_barrier()

        for b in range(RPT // CH):
            pltpu.sync_copy(acc_sh.at[pl.ds(zbase + b * CH, CH)], rows_v)
            pltpu.sync_copy(
                rows_v, parts_hbm.at[cid].at[pl.ds(zbase + b * CH, CH)])

    return sc_kernel(table, keys_pad, src, dst, zeros_blk)


BLK = 2000
GRID = N // BLK


def _tc_body(gs_ref, p0_ref, p1_ref, w_ref, m_ref, out_ref, acc_ref):
    i = pl.program_id(0)

    @pl.when(i == 0)
    def _():
        acc_ref[...] = jnp.zeros_like(acc_ref)

    dn = (((1,), (1,)), ((), ()))
    x = lax.dot_general(gs_ref[...], w_ref[...], dn,
                        preferred_element_type=jnp.float32)
    x += lax.dot_general(p0_ref[...] + p1_ref[...], m_ref[...], dn,
                         preferred_element_type=jnp.float32)
    s = jnp.maximum(x, 0.0)
    acc_ref[...] += jnp.sum(s, axis=0, keepdims=True)

    @pl.when(i == GRID - 1)
    def _():
        a = acc_ref[...]
        e = jnp.exp(a - jnp.max(a))
        out_ref[...] = e / jnp.sum(e)


def _tc_stage(gs, p0, p1, W, M):
    return pl.pallas_call(
        _tc_body,
        grid=(GRID,),
        in_specs=[
            pl.BlockSpec((BLK, D), lambda i: (i, 0)),
            pl.BlockSpec((BLK, D), lambda i: (i, 0)),
            pl.BlockSpec((BLK, D), lambda i: (i, 0)),
            pl.BlockSpec((D, D), lambda i: (0, 0)),
            pl.BlockSpec((D, D), lambda i: (0, 0)),
        ],
        out_specs=pl.BlockSpec((1, D), lambda i: (0, 0)),
        out_shape=jax.ShapeDtypeStruct((1, D), jnp.float32),
        scratch_shapes=[pltpu.VMEM((1, D), jnp.float32)],
    )(gs, p0, p1, W, M)


def kernel(index, node_keys, edge_index, W, M, Internal_Node_Impact):
    del index
    src = edge_index[0].astype(jnp.int32)
    dst = edge_index[1].astype(jnp.int32)
    keys_pad = jnp.concatenate(
        [node_keys.astype(jnp.int32), jnp.zeros((NP - N,), jnp.int32)])
    zeros_blk = jnp.zeros((ZR, D), jnp.float32)
    gs, parts = _sc_stage(Internal_Node_Impact, keys_pad, src, dst, zeros_blk)
    out = _tc_stage(gs, parts[0], parts[1], W, M)
    return out.reshape(D, 1)
